# baseline (speedup 1.0000x reference)
import jax
import jax.numpy as jnp
from jax.experimental import pallas as pl


def _copy_body(q_ref, o_ref):
    o_ref[...] = q_ref[...]


def kernel(c, q, emb_table):
    del c, emb_table
    rows, cols = q.shape
    grid = 8
    blk = rows // grid
    return pl.pallas_call(
        _copy_body,
        grid=(grid,),
        in_specs=[pl.BlockSpec((blk, cols), lambda i: (i, 0))],
        out_specs=pl.BlockSpec((blk, cols), lambda i: (i, 0)),
        out_shape=jax.ShapeDtypeStruct((rows, cols), q.dtype),
    )(q)
